# baseline (speedup 1.0000x reference)
import functools

import jax
import jax.numpy as jnp
from jax import lax
from jax.experimental import pallas as pl
from jax.experimental.pallas import tpu as pltpu
from jax.experimental.pallas import tpu_sc as plsc

B = 16
L = 2048
V = 6561
D = 768

NC = 2
NS = 16
DBLK = 128
N_BLK = D // DBLK // NC
DC = DBLK // NS
VH = 3328
HALVES = ((0, VH), (VH, V - VH))


def _gather_body(codes_hbm, cb_hbm, out_hbm, spbuf, colbuf, codes_v, outbuf,
                 osem, csem):
    c = lax.axis_index("c")
    s = lax.axis_index("s")

    splats = [jnp.full((16,), r, jnp.int32) for r in range(DC)]

    def fetch_codes(t):
        return pltpu.async_copy(
            codes_hbm.at[t % B], codes_v.at[t % 2], csem)

    codes_pending = {0: fetch_codes(0)}
    out_pending = {}
    for blk in range(N_BLK):
        col0 = (c * N_BLK + blk) * DBLK
        for r0, rn in HALVES:
            @pl.when(s == 0)
            def _():
                pltpu.sync_copy(
                    cb_hbm.at[pl.ds(r0, rn), pl.ds(col0, DBLK)],
                    spbuf.at[pl.ds(0, rn)])

            plsc.subcore_barrier()
            pltpu.sync_copy(spbuf.at[pl.ds(0, rn), pl.ds(s * DC, DC)],
                            colbuf.at[pl.ds(r0, rn)])
            plsc.subcore_barrier()

        for b in range(B):
            t = blk * B + b
            slot = t % 2
            codes_pending.pop(slot).wait()
            if t + 1 < N_BLK * B:
                codes_pending[(t + 1) % 2] = fetch_codes(t + 1)

            buf = t % 2
            if buf in out_pending:
                out_pending.pop(buf).wait()

            @plsc.parallel_loop(0, L, step=16, unroll=4)
            def _l_body(i):
                idx = codes_v[slot, pl.ds(i, 16)]
                for r in range(DC):
                    outbuf[buf, r, pl.ds(i, 16)] = plsc.load_gather(
                        colbuf, [idx, splats[r]])

            d0 = col0 + s * DC
            out_pending[buf] = pltpu.async_copy(
                outbuf.at[buf], out_hbm.at[b, pl.ds(d0, DC), :], osem)

        plsc.subcore_barrier()
    for dsc in out_pending.values():
        dsc.wait()


@jax.jit
def _features(codes, codebook):
    mesh = plsc.VectorSubcoreMesh(core_axis_name="c", subcore_axis_name="s")
    f = functools.partial(
        pl.kernel,
        out_type=jax.ShapeDtypeStruct((B, D, L), jnp.float32),
        mesh=mesh,
        scratch_types=[
            pltpu.VMEM_SHARED((VH, DBLK), jnp.float32),
            pltpu.VMEM((V, DC), jnp.float32),
            pltpu.VMEM((2, L), jnp.int32),
            pltpu.VMEM((2, DC, L), jnp.float32),
            pltpu.SemaphoreType.DMA,
            pltpu.SemaphoreType.DMA,
        ],
        compiler_params=pltpu.CompilerParams(
            use_tc_tiling_on_sc=False, needs_layout_passes=False),
    )(_gather_body)
    return f(codes, codebook)


def kernel(codes, codebook):
    return _features(codes.astype(jnp.int32), codebook)


# speedup vs baseline: 1.5629x; 1.5629x over previous
import functools

import jax
import jax.numpy as jnp
from jax import lax
from jax.experimental import pallas as pl
from jax.experimental.pallas import tpu as pltpu
from jax.experimental.pallas import tpu_sc as plsc

B = 16
L = 2048
V = 6561
D = 768

NC = 2
NS = 16
DBLK = 128
N_BLK = D // DBLK // NC
DC = DBLK // NS
VH = 3328
HALVES = ((0, VH), (VH, V - VH))


def _gather_body(codes_hbm, cb_hbm, out_hbm, spbuf, colbuf, codes_v, outbuf,
                 osem, csem):
    c = lax.axis_index("c")
    s = lax.axis_index("s")

    splats = [jnp.full((16,), r, jnp.int32) for r in range(DC)]

    def fetch_codes(t):
        return pltpu.async_copy(
            codes_hbm.at[t % B], codes_v.at[t % 2], csem)

    codes_pending = {0: fetch_codes(0)}
    out_pending = {}
    for blk in range(N_BLK):
        col0 = (c * N_BLK + blk) * DBLK
        for r0, rn in HALVES:
            @pl.when(s == 0)
            def _():
                pltpu.sync_copy(
                    cb_hbm.at[pl.ds(r0, rn), pl.ds(col0, DBLK)],
                    spbuf.at[pl.ds(0, rn)])

            plsc.subcore_barrier()
            pltpu.sync_copy(spbuf.at[pl.ds(0, rn), pl.ds(s * DC, DC)],
                            colbuf.at[pl.ds(r0, rn)])
            plsc.subcore_barrier()

        for b in range(B):
            t = blk * B + b
            slot = t % 2
            codes_pending.pop(slot).wait()
            if t + 1 < N_BLK * B:
                codes_pending[(t + 1) % 2] = fetch_codes(t + 1)

            buf = t % 2
            if buf in out_pending:
                out_pending.pop(buf).wait()

            @plsc.parallel_loop(0, L, step=16, unroll=4)
            def _l_body(i):
                idx = codes_v[slot, pl.ds(i, 16)]
                q = lax.shift_right_logical(i, 7)
                m = lax.bitwise_and(i, 127)
                for r in range(DC):
                    outbuf[buf, q, r, pl.ds(m, 16)] = plsc.load_gather(
                        colbuf, [idx, splats[r]])

            dblk = (col0 + s * DC) // DC
            out_pending[buf] = pltpu.async_copy(
                outbuf.at[buf], out_hbm.at[b, dblk], osem)

        plsc.subcore_barrier()
    for dsc in out_pending.values():
        dsc.wait()


@jax.jit
def _features(codes, codebook):
    mesh = plsc.VectorSubcoreMesh(core_axis_name="c", subcore_axis_name="s")
    f = functools.partial(
        pl.kernel,
        out_type=jax.ShapeDtypeStruct((B, D // DC, L // DBLK, DC, DBLK),
                                      jnp.float32),
        mesh=mesh,
        scratch_types=[
            pltpu.VMEM_SHARED((VH, DBLK), jnp.float32),
            pltpu.VMEM((V, DC), jnp.float32),
            pltpu.VMEM((2, L), jnp.int32),
            pltpu.VMEM((2, L // DBLK, DC, DBLK), jnp.float32),
            pltpu.SemaphoreType.DMA,
            pltpu.SemaphoreType.DMA,
        ],
        compiler_params=pltpu.CompilerParams(
            use_tc_tiling_on_sc=False, needs_layout_passes=False),
    )(_gather_body)
    out5 = f(codes, codebook)
    return out5.transpose(0, 1, 3, 2, 4).reshape(B, D, L)


def kernel(codes, codebook):
    return _features(codes.astype(jnp.int32), codebook)


# speedup vs baseline: 1.5935x; 1.0196x over previous
import functools

import jax
import jax.numpy as jnp
from jax import lax
from jax.experimental import pallas as pl
from jax.experimental.pallas import tpu as pltpu
from jax.experimental.pallas import tpu_sc as plsc

B = 16
L = 2048
V = 6561
D = 768

NC = 2
NS = 16
DBLK = 128
N_BLK = D // DBLK // NC
DC = DBLK // NS
VH = 3328
HALVES = ((0, VH), (VH, V - VH))


def _gather_body(codes_hbm, cb_hbm, out_hbm, colbuf, codes_v, outbuf,
                 osem, csem):
    c = lax.axis_index("c")
    s = lax.axis_index("s")

    splats = [jnp.full((16,), r, jnp.int32) for r in range(DC)]

    def fetch_codes(t):
        return pltpu.async_copy(
            codes_hbm.at[t % B], codes_v.at[t % 2], csem)

    codes_pending = {0: fetch_codes(0)}
    out_pending = {}
    for blk in range(N_BLK):
        col0 = (c * N_BLK + blk) * DBLK
        pltpu.sync_copy(cb_hbm.at[:, pl.ds(col0 + s * DC, DC)], colbuf)

        for b in range(B):
            t = blk * B + b
            slot = t % 2
            codes_pending.pop(slot).wait()
            if t + 1 < N_BLK * B:
                codes_pending[(t + 1) % 2] = fetch_codes(t + 1)

            buf = t % 2
            if buf in out_pending:
                out_pending.pop(buf).wait()

            @plsc.parallel_loop(0, L, step=16, unroll=4)
            def _l_body(i):
                idx = codes_v[slot, pl.ds(i, 16)]
                q = lax.shift_right_logical(i, 7)
                m = lax.bitwise_and(i, 127)
                for r in range(DC):
                    outbuf[buf, q, r, pl.ds(m, 16)] = plsc.load_gather(
                        colbuf, [idx, splats[r]])

            dblk = (col0 + s * DC) // DC
            out_pending[buf] = pltpu.async_copy(
                outbuf.at[buf], out_hbm.at[b, dblk], osem)

    for dsc in out_pending.values():
        dsc.wait()


@jax.jit
def _features(codes, codebook):
    mesh = plsc.VectorSubcoreMesh(core_axis_name="c", subcore_axis_name="s")
    f = functools.partial(
        pl.kernel,
        out_type=jax.ShapeDtypeStruct((B, D // DC, L // DBLK, DC, DBLK),
                                      jnp.float32),
        mesh=mesh,
        scratch_types=[
            pltpu.VMEM((V, DC), jnp.float32),
            pltpu.VMEM((2, L), jnp.int32),
            pltpu.VMEM((2, L // DBLK, DC, DBLK), jnp.float32),
            pltpu.SemaphoreType.DMA,
            pltpu.SemaphoreType.DMA,
        ],
        compiler_params=pltpu.CompilerParams(
            use_tc_tiling_on_sc=False, needs_layout_passes=False),
    )(_gather_body)
    out5 = f(codes, codebook)
    return out5.transpose(0, 1, 3, 2, 4).reshape(B, D, L)


def kernel(codes, codebook):
    return _features(codes.astype(jnp.int32), codebook)


# speedup vs baseline: 1.9362x; 1.2150x over previous
import functools

import jax
import jax.numpy as jnp
from jax import lax
from jax.experimental import pallas as pl
from jax.experimental.pallas import tpu as pltpu
from jax.experimental.pallas import tpu_sc as plsc

B = 16
L = 2048
V = 6561
VP = 6656
D = 768

NC = 2
NS = 16
DBLK = 128
N_BLK = D // DBLK // NC
DC = DBLK // NS
VB = VP // 128
DB = D // DC
LB = L // DBLK


def _t4_body(cb_ref, out_ref):
    x = cb_ref[...]
    out_ref[...] = x.T.reshape(DB, DC, DBLK)[:, None]


def _gather_body(codes_hbm, t4_hbm, out_hbm, colbuf, codes_v, outbuf,
                 osem, csem):
    c = lax.axis_index("c")
    s = lax.axis_index("s")

    splats = [jnp.full((16,), r, jnp.int32) for r in range(DC)]

    def fetch_codes(t):
        return pltpu.async_copy(
            codes_hbm.at[t % B], codes_v.at[t % 2], csem)

    codes_pending = {0: fetch_codes(0)}
    out_pending = {}
    for blk in range(N_BLK):
        db = (c * N_BLK + blk) * NS + s
        pltpu.sync_copy(t4_hbm.at[db], colbuf)

        for b in range(B):
            t = blk * B + b
            slot = t % 2
            codes_pending.pop(slot).wait()
            if t + 1 < N_BLK * B:
                codes_pending[(t + 1) % 2] = fetch_codes(t + 1)

            buf = t % 2
            if buf in out_pending:
                out_pending.pop(buf).wait()

            @plsc.parallel_loop(0, L, step=16, unroll=4)
            def _l_body(i):
                idx = codes_v[slot, pl.ds(i, 16)]
                vb = lax.shift_right_logical(idx, 7)
                vl = lax.bitwise_and(idx, 127)
                q = lax.shift_right_logical(i, 7)
                m = lax.bitwise_and(i, 127)
                for r in range(DC):
                    outbuf[buf, q, r, pl.ds(m, 16)] = plsc.load_gather(
                        colbuf, [vb, splats[r], vl])

            out_pending[buf] = pltpu.async_copy(
                outbuf.at[buf], out_hbm.at[b, db], osem)

    for dsc in out_pending.values():
        dsc.wait()


@jax.jit
def _features(codes, codebook):
    t4 = pl.pallas_call(
        _t4_body,
        grid=(VB,),
        in_specs=[pl.BlockSpec((DBLK, D), lambda i: (i, 0))],
        out_specs=pl.BlockSpec((DB, 1, DC, DBLK), lambda i: (0, i, 0, 0)),
        out_shape=jax.ShapeDtypeStruct((DB, VB, DC, DBLK), jnp.float32),
    )(codebook)

    mesh = plsc.VectorSubcoreMesh(core_axis_name="c", subcore_axis_name="s")
    f = functools.partial(
        pl.kernel,
        out_type=jax.ShapeDtypeStruct((B, DB, LB, DC, DBLK), jnp.float32),
        mesh=mesh,
        scratch_types=[
            pltpu.VMEM((VB, DC, DBLK), jnp.float32),
            pltpu.VMEM((2, L), jnp.int32),
            pltpu.VMEM((2, LB, DC, DBLK), jnp.float32),
            pltpu.SemaphoreType.DMA,
            pltpu.SemaphoreType.DMA,
        ],
        compiler_params=pltpu.CompilerParams(
            use_tc_tiling_on_sc=False, needs_layout_passes=False),
    )(_gather_body)
    out5 = f(codes, t4)
    return out5.transpose(0, 1, 3, 2, 4).reshape(B, D, L)


def kernel(codes, codebook):
    return _features(codes.astype(jnp.int32), codebook)


# speedup vs baseline: 2.2506x; 1.1623x over previous
import functools

import jax
import jax.numpy as jnp
from jax import lax
from jax.experimental import pallas as pl
from jax.experimental.pallas import tpu as pltpu
from jax.experimental.pallas import tpu_sc as plsc

B = 16
L = 2048
V = 6561
VP = 6656
D = 768

NC = 2
NS = 16
DBLK = 128
N_BLK = D // DBLK // NC
DC = DBLK // NS
VB = VP // 128
DB = D // DC
LB = L // DBLK


BR = 512
JB = BR // DBLK


def _t4_body(cb_ref, out_ref):
    for j in range(JB):
        x = cb_ref[pl.ds(j * DBLK, DBLK), :]
        out_ref[:, j] = x.T.reshape(DB, DC, DBLK)


def _gather_body(codes_hbm, t4_hbm, out_hbm, colbuf, codes_v, outbuf,
                 osem, csem):
    c = lax.axis_index("c")
    s = lax.axis_index("s")

    splats = [jnp.full((16,), r, jnp.int32) for r in range(DC)]

    def fetch_codes(t):
        return pltpu.async_copy(
            codes_hbm.at[t % B], codes_v.at[t % 2], csem)

    codes_pending = {0: fetch_codes(0)}
    out_pending = {}
    for blk in range(N_BLK):
        db = (c * N_BLK + blk) * NS + s
        pltpu.sync_copy(t4_hbm.at[db], colbuf)

        for b in range(B):
            t = blk * B + b
            slot = t % 2
            codes_pending.pop(slot).wait()
            if t + 1 < N_BLK * B:
                codes_pending[(t + 1) % 2] = fetch_codes(t + 1)

            buf = t % 2
            if buf in out_pending:
                out_pending.pop(buf).wait()

            @plsc.parallel_loop(0, L, step=16, unroll=4)
            def _l_body(i):
                idx = codes_v[slot, pl.ds(i, 16)]
                vb = lax.shift_right_logical(idx, 7)
                vl = lax.bitwise_and(idx, 127)
                q = lax.shift_right_logical(i, 7)
                m = lax.bitwise_and(i, 127)
                for r in range(DC):
                    outbuf[buf, q, r, pl.ds(m, 16)] = plsc.load_gather(
                        colbuf, [vb, splats[r], vl])

            out_pending[buf] = pltpu.async_copy(
                outbuf.at[buf], out_hbm.at[b, db], osem)

    for dsc in out_pending.values():
        dsc.wait()


@jax.jit
def _features(codes, codebook):
    t4 = pl.pallas_call(
        _t4_body,
        grid=(VB // JB,),
        in_specs=[pl.BlockSpec((BR, D), lambda i: (i, 0))],
        out_specs=pl.BlockSpec((DB, JB, DC, DBLK), lambda i: (0, i, 0, 0)),
        out_shape=jax.ShapeDtypeStruct((DB, VB, DC, DBLK), jnp.float32),
    )(codebook)

    mesh = plsc.VectorSubcoreMesh(core_axis_name="c", subcore_axis_name="s")
    f = functools.partial(
        pl.kernel,
        out_type=jax.ShapeDtypeStruct((B, DB, LB, DC, DBLK), jnp.float32),
        mesh=mesh,
        scratch_types=[
            pltpu.VMEM((VB, DC, DBLK), jnp.float32),
            pltpu.VMEM((2, L), jnp.int32),
            pltpu.VMEM((2, LB, DC, DBLK), jnp.float32),
            pltpu.SemaphoreType.DMA,
            pltpu.SemaphoreType.DMA,
        ],
        compiler_params=pltpu.CompilerParams(
            use_tc_tiling_on_sc=False, needs_layout_passes=False),
    )(_gather_body)
    out5 = f(codes, t4)
    return out5.transpose(0, 1, 3, 2, 4).reshape(B, D, L)


def kernel(codes, codebook):
    return _features(codes.astype(jnp.int32), codebook)


# speedup vs baseline: 2.3796x; 1.0573x over previous
import functools

import jax
import jax.numpy as jnp
from jax import lax
from jax.experimental import pallas as pl
from jax.experimental.pallas import tpu as pltpu
from jax.experimental.pallas import tpu_sc as plsc

B = 16
L = 2048
V = 6561
VP = 6656
D = 768

NC = 2
NS = 16
DBLK = 128
N_BLK = D // DBLK // NC
DC = DBLK // NS
VB = VP // 128
DB = D // DC
LB = L // DBLK


BR = 512
JB = BR // DBLK


def _t4_body(cb_ref, out_ref):
    for j in range(JB):
        x = cb_ref[pl.ds(j * DBLK, DBLK), :]
        out_ref[:, j] = x.T.reshape(DB, DC, DBLK)


LH = L // 2
QH = LH // DBLK


def _gather_body(codes_hbm, t4_hbm, out_hbm, colbuf, codes_v, outbuf,
                 osem, csem, ssem):
    c = lax.axis_index("c")
    s = lax.axis_index("s")

    splats = [jnp.full((16,), r, jnp.int32) for r in range(DC)]

    def db_of(blk):
        return (c * N_BLK + blk) * NS + s

    def fetch_slice(blk):
        return pltpu.async_copy(
            t4_hbm.at[db_of(blk)], colbuf.at[blk % 2], ssem)

    pltpu.async_copy(codes_hbm.at[0], codes_v.at[0], csem)
    stage_pending = {0: fetch_slice(0)}
    for blk in range(N_BLK):
        cslot = blk % 2
        stage_pending.pop(cslot).wait()
        if blk + 1 < N_BLK:
            stage_pending[(blk + 1) % 2] = fetch_slice(blk + 1)
        cref = colbuf.at[cslot]
        db = db_of(blk)

        def b_body(b, _, blk=blk, cref=cref, db=db):
            slot = lax.bitwise_and(b, 1)
            pltpu.make_async_copy(
                codes_hbm.at[b], codes_v.at[slot], csem).wait()
            last = (blk == N_BLK - 1)

            @pl.when((b < B - 1) if last else (b < B))
            def _():
                pltpu.async_copy(codes_hbm.at[lax.bitwise_and(b + 1, B - 1)],
                                 codes_v.at[1 - slot], csem)

            for h in range(2):
                def wait_out():
                    pltpu.make_async_copy(
                        outbuf.at[h],
                        out_hbm.at[b, db, pl.ds(h * QH, QH)], osem).wait()

                if blk == 0:
                    pl.when(b > 0)(wait_out)
                else:
                    wait_out()

                @plsc.parallel_loop(0, LH, step=16, unroll=4)
                def _l_body(i):
                    idx = codes_v[slot, pl.ds(h * LH + i, 16)]
                    vb = lax.shift_right_logical(idx, 7)
                    vl = lax.bitwise_and(idx, 127)
                    q = lax.shift_right_logical(i, 7)
                    m = lax.bitwise_and(i, 127)
                    for r in range(DC):
                        outbuf[h, q, r, pl.ds(m, 16)] = plsc.load_gather(
                            cref, [vb, splats[r], vl])

                pltpu.async_copy(
                    outbuf.at[h],
                    out_hbm.at[b, db, pl.ds(h * QH, QH)], osem)
            return 0

        lax.fori_loop(0, B, b_body, 0)

    for h in range(2):
        pltpu.make_async_copy(
            outbuf.at[h],
            out_hbm.at[B - 1, db_of(N_BLK - 1), pl.ds(h * QH, QH)],
            osem).wait()


@jax.jit
def _features(codes, codebook):
    t4 = pl.pallas_call(
        _t4_body,
        grid=(VB // JB,),
        in_specs=[pl.BlockSpec((BR, D), lambda i: (i, 0))],
        out_specs=pl.BlockSpec((DB, JB, DC, DBLK), lambda i: (0, i, 0, 0)),
        out_shape=jax.ShapeDtypeStruct((DB, VB, DC, DBLK), jnp.float32),
    )(codebook)

    mesh = plsc.VectorSubcoreMesh(core_axis_name="c", subcore_axis_name="s")
    f = functools.partial(
        pl.kernel,
        out_type=jax.ShapeDtypeStruct((B, DB, LB, DC, DBLK), jnp.float32),
        mesh=mesh,
        scratch_types=[
            pltpu.VMEM((2, VB, DC, DBLK), jnp.float32),
            pltpu.VMEM((2, L), jnp.int32),
            pltpu.VMEM((2, QH, DC, DBLK), jnp.float32),
            pltpu.SemaphoreType.DMA,
            pltpu.SemaphoreType.DMA,
            pltpu.SemaphoreType.DMA,
        ],
        compiler_params=pltpu.CompilerParams(
            use_tc_tiling_on_sc=False, needs_layout_passes=False),
    )(_gather_body)
    out5 = f(codes, t4)
    return out5.transpose(0, 1, 3, 2, 4).reshape(B, D, L)


def kernel(codes, codebook):
    return _features(codes.astype(jnp.int32), codebook)


# speedup vs baseline: 2.4386x; 1.0248x over previous
import functools

import jax
import jax.numpy as jnp
from jax import lax
from jax.experimental import pallas as pl
from jax.experimental.pallas import tpu as pltpu
from jax.experimental.pallas import tpu_sc as plsc

B = 16
L = 2048
V = 6561
VP = 6656
D = 768

NC = 2
NS = 16
DBLK = 128
N_BLK = D // DBLK // NC
DC = DBLK // NS
VB = VP // 128
DB = D // DC
LB = L // DBLK


BR = 1664
JB = BR // DBLK


def _t4_body(cb_ref, out_ref):
    for j in range(JB):
        x = cb_ref[pl.ds(j * DBLK, DBLK), :]
        out_ref[:, j] = x.T.reshape(DB, DC, DBLK)


LH = L // 2
QH = LH // DBLK


def _gather_body(codes_hbm, t4_hbm, out_hbm, colbuf, codes_v, outbuf,
                 osem, csem, ssem):
    c = lax.axis_index("c")
    s = lax.axis_index("s")

    splats = [jnp.full((16,), r, jnp.int32) for r in range(DC)]

    def db_of(blk):
        return (c * N_BLK + blk) * NS + s

    def fetch_slice(blk):
        return pltpu.async_copy(
            t4_hbm.at[db_of(blk)], colbuf.at[blk % 2], ssem)

    pltpu.async_copy(codes_hbm.at[0], codes_v.at[0], csem)
    stage_pending = {0: fetch_slice(0)}
    for blk in range(N_BLK):
        cslot = blk % 2
        stage_pending.pop(cslot).wait()
        if blk + 1 < N_BLK:
            stage_pending[(blk + 1) % 2] = fetch_slice(blk + 1)
        cref = colbuf.at[cslot]
        db = db_of(blk)

        def b_body(b, _, blk=blk, cref=cref, db=db):
            slot = lax.bitwise_and(b, 1)
            pltpu.make_async_copy(
                codes_hbm.at[b], codes_v.at[slot], csem).wait()
            last = (blk == N_BLK - 1)

            @pl.when((b < B - 1) if last else (b < B))
            def _():
                pltpu.async_copy(codes_hbm.at[lax.bitwise_and(b + 1, B - 1)],
                                 codes_v.at[1 - slot], csem)

            for h in range(2):
                def wait_out():
                    pltpu.make_async_copy(
                        outbuf.at[h],
                        out_hbm.at[b, db, pl.ds(h * QH, QH)], osem).wait()

                if blk == 0:
                    pl.when(b > 0)(wait_out)
                else:
                    wait_out()

                @plsc.parallel_loop(0, LH, step=16, unroll=4)
                def _l_body(i):
                    idx = codes_v[slot, pl.ds(h * LH + i, 16)]
                    vb = lax.shift_right_logical(idx, 7)
                    vl = lax.bitwise_and(idx, 127)
                    q = lax.shift_right_logical(i, 7)
                    m = lax.bitwise_and(i, 127)
                    for r in range(DC):
                        outbuf[h, q, r, pl.ds(m, 16)] = plsc.load_gather(
                            cref, [vb, splats[r], vl])

                pltpu.async_copy(
                    outbuf.at[h],
                    out_hbm.at[b, db, pl.ds(h * QH, QH)], osem)
            return 0

        lax.fori_loop(0, B, b_body, 0)

    for h in range(2):
        pltpu.make_async_copy(
            outbuf.at[h],
            out_hbm.at[B - 1, db_of(N_BLK - 1), pl.ds(h * QH, QH)],
            osem).wait()


@jax.jit
def _features(codes, codebook):
    t4 = pl.pallas_call(
        _t4_body,
        grid=(VB // JB,),
        in_specs=[pl.BlockSpec((BR, D), lambda i: (i, 0))],
        out_specs=pl.BlockSpec((DB, JB, DC, DBLK), lambda i: (0, i, 0, 0)),
        out_shape=jax.ShapeDtypeStruct((DB, VB, DC, DBLK), jnp.float32),
    )(codebook)

    mesh = plsc.VectorSubcoreMesh(core_axis_name="c", subcore_axis_name="s")
    f = functools.partial(
        pl.kernel,
        out_type=jax.ShapeDtypeStruct((B, DB, LB, DC, DBLK), jnp.float32),
        mesh=mesh,
        scratch_types=[
            pltpu.VMEM((2, VB, DC, DBLK), jnp.float32),
            pltpu.VMEM((2, L), jnp.int32),
            pltpu.VMEM((2, QH, DC, DBLK), jnp.float32),
            pltpu.SemaphoreType.DMA,
            pltpu.SemaphoreType.DMA,
            pltpu.SemaphoreType.DMA,
        ],
        compiler_params=pltpu.CompilerParams(
            use_tc_tiling_on_sc=False, needs_layout_passes=False),
    )(_gather_body)
    out5 = f(codes, t4)
    return out5.transpose(0, 1, 3, 2, 4).reshape(B, D, L)


def kernel(codes, codebook):
    return _features(codes.astype(jnp.int32), codebook)


# speedup vs baseline: 2.4881x; 1.0203x over previous
import functools

import jax
import jax.numpy as jnp
from jax import lax
from jax.experimental import pallas as pl
from jax.experimental.pallas import tpu as pltpu
from jax.experimental.pallas import tpu_sc as plsc

B = 16
L = 2048
V = 6561
VP = 6656
D = 768

NC = 2
NS = 16
DBLK = 128
N_BLK = D // DBLK // NC
DC = DBLK // NS
VB = VP // 128
DB = D // DC
LB = L // DBLK


BR = 3328
JB = BR // DBLK


def _t4_body(cb_ref, out_ref):
    for j in range(JB):
        x = cb_ref[pl.ds(j * DBLK, DBLK), :]
        out_ref[:, j] = x.T.reshape(DB, DC, DBLK)


LH = L // 2
QH = LH // DBLK


def _gather_body(codes_hbm, t4_hbm, out_hbm, colbuf, codes_v, outbuf,
                 osem, csem, ssem):
    c = lax.axis_index("c")
    s = lax.axis_index("s")

    splats = [jnp.full((16,), r, jnp.int32) for r in range(DC)]

    def db_of(blk):
        return (c * N_BLK + blk) * NS + s

    def fetch_slice(blk):
        return pltpu.async_copy(
            t4_hbm.at[db_of(blk)], colbuf.at[blk % 2], ssem)

    pltpu.async_copy(codes_hbm.at[0], codes_v.at[0], csem)
    stage_pending = {0: fetch_slice(0)}
    for blk in range(N_BLK):
        cslot = blk % 2
        stage_pending.pop(cslot).wait()
        if blk + 1 < N_BLK:
            stage_pending[(blk + 1) % 2] = fetch_slice(blk + 1)
        cref = colbuf.at[cslot]
        db = db_of(blk)

        def b_body(b, _, blk=blk, cref=cref, db=db):
            slot = lax.bitwise_and(b, 1)
            pltpu.make_async_copy(
                codes_hbm.at[b], codes_v.at[slot], csem).wait()
            last = (blk == N_BLK - 1)

            @pl.when((b < B - 1) if last else (b < B))
            def _():
                pltpu.async_copy(codes_hbm.at[lax.bitwise_and(b + 1, B - 1)],
                                 codes_v.at[1 - slot], csem)

            for h in range(2):
                def wait_out():
                    pltpu.make_async_copy(
                        outbuf.at[h],
                        out_hbm.at[b, db, pl.ds(h * QH, QH)], osem).wait()

                if blk == 0:
                    pl.when(b > 0)(wait_out)
                else:
                    wait_out()

                @plsc.parallel_loop(0, LH, step=16, unroll=4)
                def _l_body(i):
                    idx = codes_v[slot, pl.ds(h * LH + i, 16)]
                    vb = lax.shift_right_logical(idx, 7)
                    vl = lax.bitwise_and(idx, 127)
                    q = lax.shift_right_logical(i, 7)
                    m = lax.bitwise_and(i, 127)
                    for r in range(DC):
                        outbuf[h, q, r, pl.ds(m, 16)] = plsc.load_gather(
                            cref, [vb, splats[r], vl])

                pltpu.async_copy(
                    outbuf.at[h],
                    out_hbm.at[b, db, pl.ds(h * QH, QH)], osem)
            return 0

        lax.fori_loop(0, B, b_body, 0)

    for h in range(2):
        pltpu.make_async_copy(
            outbuf.at[h],
            out_hbm.at[B - 1, db_of(N_BLK - 1), pl.ds(h * QH, QH)],
            osem).wait()


@jax.jit
def _features(codes, codebook):
    t4 = pl.pallas_call(
        _t4_body,
        grid=(VB // JB,),
        in_specs=[pl.BlockSpec((BR, D), lambda i: (i, 0))],
        out_specs=pl.BlockSpec((DB, JB, DC, DBLK), lambda i: (0, i, 0, 0)),
        out_shape=jax.ShapeDtypeStruct((DB, VB, DC, DBLK), jnp.float32),
    )(codebook)

    mesh = plsc.VectorSubcoreMesh(core_axis_name="c", subcore_axis_name="s")
    f = functools.partial(
        pl.kernel,
        out_type=jax.ShapeDtypeStruct((B, DB, LB, DC, DBLK), jnp.float32),
        mesh=mesh,
        scratch_types=[
            pltpu.VMEM((2, VB, DC, DBLK), jnp.float32),
            pltpu.VMEM((2, L), jnp.int32),
            pltpu.VMEM((2, QH, DC, DBLK), jnp.float32),
            pltpu.SemaphoreType.DMA,
            pltpu.SemaphoreType.DMA,
            pltpu.SemaphoreType.DMA,
        ],
        compiler_params=pltpu.CompilerParams(
            use_tc_tiling_on_sc=False, needs_layout_passes=False),
    )(_gather_body)
    out5 = f(codes, t4)
    return out5.transpose(0, 1, 3, 2, 4).reshape(B, D, L)


def kernel(codes, codebook):
    return _features(codes.astype(jnp.int32), codebook)
